# Initial kernel scaffold; baseline (speedup 1.0000x reference)
import functools

import jax
import jax.numpy as jnp
from jax import lax
from jax.experimental import pallas as pl
from jax.experimental.pallas import tpu as pltpu
from jax.experimental.pallas import tpu_sc as plsc

N = 10000
E = 320000
D = 128
H = 64
B = 64

NC = 2
NS = 16
NW = NC * NS

K = 128
C = 79
EPAD = NW * C * K - E
NPAD = N + 16
RPT = NPAD // NS

_MESH = plsc.VectorSubcoreMesh(core_axis_name="c", subcore_axis_name="s")


@functools.partial(
    pl.kernel,
    mesh=_MESH,
    out_type=jax.ShapeDtypeStruct((NC, NPAD, H), jnp.float32),
    scratch_types=[
        pltpu.VMEM((C, K), jnp.int32),
        pltpu.VMEM((C, K), jnp.int32),
        pltpu.VMEM((K, H), jnp.float32),
        pltpu.VMEM_SHARED((NPAD, H), jnp.float32),
    ],
)
def _agg_kernel(q_hbm, src_hbm, dst_hbm, zeros_hbm, out_hbm,
                src_v, dst_v, rows_v, acc_sh):
    c = lax.axis_index("c")
    s = lax.axis_index("s")
    wid = s * NC + c

    pltpu.sync_copy(src_hbm.at[wid], src_v)
    pltpu.sync_copy(dst_hbm.at[wid], dst_v)
    pltpu.sync_copy(zeros_hbm.at[pl.ds(s * RPT, RPT)],
                    acc_sh.at[pl.ds(s * RPT, RPT)])
    plsc.subcore_barrier()

    def body(j, carry):
        pltpu.sync_copy(q_hbm.at[src_v.at[j]], rows_v)
        pltpu.sync_copy(rows_v, acc_sh.at[dst_v.at[j]], add=True)
        return carry

    lax.fori_loop(0, C, body, 0)

    plsc.subcore_barrier()
    pltpu.sync_copy(acc_sh.at[pl.ds(s * RPT, RPT)],
                    out_hbm.at[c, pl.ds(s * RPT, RPT)])


def _mm_body(x_ref, w_ref, o_ref):
    o_ref[...] = jnp.dot(x_ref[...], w_ref[...],
                         preferred_element_type=jnp.float32)


def _layer_tail(q, parts, ba, g, be, Wb, bb, Wna, batch2):
    def body(q_ref, p_ref, ba_ref, g_ref, be_ref, wb_ref, bb_ref, wna_ref,
             batch_ref, pool_ref, qn_ref):
        h2 = q_ref[...] + p_ref[0, :N, :] + p_ref[1, :N, :] + ba_ref[...]
        m = jnp.mean(h2, axis=0, keepdims=True)
        v = jnp.mean((h2 - m) * (h2 - m), axis=0, keepdims=True)
        hn = (h2 - m) * lax.rsqrt(v + 1e-5) * g_ref[...] + be_ref[...]
        hn = jnp.maximum(hn, 0.0)
        xo = jnp.dot(hn, wb_ref[...], preferred_element_type=jnp.float32)
        xo = xo + bb_ref[...]
        ids = lax.broadcasted_iota(jnp.int32, (B, N), 0)
        oh = (batch_ref[...] == ids).astype(jnp.float32)
        cnt = jnp.maximum(jnp.sum(oh, axis=1, keepdims=True), 1.0)
        pool_ref[...] = jnp.dot(oh, xo, preferred_element_type=jnp.float32) / cnt
        qn_ref[...] = jnp.dot(xo, wna_ref[...],
                              preferred_element_type=jnp.float32)

    return pl.pallas_call(
        body,
        out_shape=[jax.ShapeDtypeStruct((B, H), jnp.float32),
                   jax.ShapeDtypeStruct((N, H), jnp.float32)],
    )(q, parts, ba, g, be, Wb, bb, Wna, batch2)


def _final(q, parts, ba, g, be, Wb, bb, batch2, pool1, pool2,
           lin1_W, lin1_b, lin2_W, lin2_b):
    def body(q_ref, p_ref, ba_ref, g_ref, be_ref, wb_ref, bb_ref, batch_ref,
             p1_ref, p2_ref, l1w_ref, l1b_ref, l2w_ref, l2b_ref, o_ref):
        h2 = q_ref[...] + p_ref[0, :N, :] + p_ref[1, :N, :] + ba_ref[...]
        m = jnp.mean(h2, axis=0, keepdims=True)
        v = jnp.mean((h2 - m) * (h2 - m), axis=0, keepdims=True)
        hn = (h2 - m) * lax.rsqrt(v + 1e-5) * g_ref[...] + be_ref[...]
        hn = jnp.maximum(hn, 0.0)
        xo = jnp.dot(hn, wb_ref[...], preferred_element_type=jnp.float32)
        xo = xo + bb_ref[...]
        ids = lax.broadcasted_iota(jnp.int32, (B, N), 0)
        oh = (batch_ref[...] == ids).astype(jnp.float32)
        cnt = jnp.maximum(jnp.sum(oh, axis=1, keepdims=True), 1.0)
        pool3 = jnp.dot(oh, xo, preferred_element_type=jnp.float32) / cnt
        z = jnp.concatenate([p1_ref[...], p2_ref[...], pool3], axis=1)
        zz = jnp.dot(z, l1w_ref[...], preferred_element_type=jnp.float32)
        zz = jnp.maximum(zz + l1b_ref[...], 0.0)
        o_ref[...] = jnp.dot(zz, l2w_ref[...],
                             preferred_element_type=jnp.float32) + l2b_ref[...]

    return pl.pallas_call(
        body,
        out_shape=jax.ShapeDtypeStruct((B, 1), jnp.float32),
    )(q, parts, ba, g, be, Wb, bb, batch2, pool1, pool2,
      lin1_W, lin1_b, lin2_W, lin2_b)


def kernel(x, edge_index, batch,
           W1a, b1a, g1, be1, W1b, b1b,
           W2a, b2a, g2, be2, W2b, b2b,
           W3a, b3a, g3, be3, W3b, b3b,
           lin1_W, lin1_b, lin2_W, lin2_b):
    src = jnp.concatenate(
        [edge_index[0], jnp.zeros((EPAD,), jnp.int32)]).reshape(NW, C, K)
    dst = jnp.concatenate(
        [edge_index[1], jnp.full((EPAD,), N, jnp.int32)]).reshape(NW, C, K)
    zeros = jnp.zeros((NPAD, H), jnp.float32)
    batch2 = batch.reshape(1, N)

    def agg(q):
        return _agg_kernel(q, src, dst, zeros)

    r = lambda a: a.reshape(1, -1)

    q1 = pl.pallas_call(
        _mm_body, out_shape=jax.ShapeDtypeStruct((N, H), jnp.float32))(x, W1a)
    pool1, q2 = _layer_tail(q1, agg(q1), r(b1a), r(g1), r(be1), W1b, r(b1b),
                            W2a, batch2)
    pool2, q3 = _layer_tail(q2, agg(q2), r(b2a), r(g2), r(be2), W2b, r(b2b),
                            W3a, batch2)
    return _final(q3, agg(q3), r(b3a), r(g3), r(be3), W3b, r(b3b), batch2,
                  pool1, pool2, lin1_W, r(lin1_b), lin2_W, r(lin2_b))


# speedup vs baseline: 3.7957x; 3.7957x over previous
import functools

import jax
import jax.numpy as jnp
from jax import lax
from jax.experimental import pallas as pl
from jax.experimental.pallas import tpu as pltpu
from jax.experimental.pallas import tpu_sc as plsc

N = 10000
E = 320000
D = 128
H = 64
B = 64

NC = 2
NS = 16
NW = NC * NS

HP = 128
K = 128
C = 79
EPAD = NW * C * K - E
NPAD = N + 112
RPT = NPAD // NS

_MESH = plsc.VectorSubcoreMesh(core_axis_name="c", subcore_axis_name="s")


@functools.partial(
    pl.kernel,
    mesh=_MESH,
    out_type=jax.ShapeDtypeStruct((NC, NPAD, HP), jnp.float32),
    scratch_types=[
        pltpu.VMEM((C, K), jnp.int32),
        pltpu.VMEM((C, K), jnp.int32),
        pltpu.VMEM((K, HP), jnp.float32),
        pltpu.VMEM_SHARED((NPAD, HP), jnp.float32),
    ],
)
def _agg_kernel(q_hbm, src_hbm, dst_hbm, zeros_hbm, out_hbm,
                src_v, dst_v, rows_v, acc_sh):
    c = lax.axis_index("c")
    s = lax.axis_index("s")
    wid = s * NC + c

    pltpu.sync_copy(src_hbm.at[wid], src_v)
    pltpu.sync_copy(dst_hbm.at[wid], dst_v)
    pltpu.sync_copy(zeros_hbm.at[pl.ds(s * RPT, RPT)],
                    acc_sh.at[pl.ds(s * RPT, RPT)])
    plsc.subcore_barrier()

    def body(j, carry):
        pltpu.sync_copy(q_hbm.at[src_v.at[j]], rows_v)
        pltpu.sync_copy(rows_v, acc_sh.at[dst_v.at[j]], add=True)
        return carry

    lax.fori_loop(0, C, body, 0)

    plsc.subcore_barrier()
    pltpu.sync_copy(acc_sh.at[pl.ds(s * RPT, RPT)],
                    out_hbm.at[c, pl.ds(s * RPT, RPT)])


def _mm_body(x_ref, w_ref, o_ref):
    o_ref[...] = jnp.dot(x_ref[...], w_ref[...],
                         preferred_element_type=jnp.float32)


def _layer_tail(q, parts, ba, g, be, Wb, bb, Wna, batch2):
    def body(q_ref, p_ref, ba_ref, g_ref, be_ref, wb_ref, bb_ref, wna_ref,
             batch_ref, pool_ref, qn_ref):
        h2 = (q_ref[...] + p_ref[0, :N, :] + p_ref[1, :N, :])[:, :H]
        h2 = h2 + ba_ref[...]
        m = jnp.mean(h2, axis=0, keepdims=True)
        v = jnp.mean((h2 - m) * (h2 - m), axis=0, keepdims=True)
        hn = (h2 - m) * lax.rsqrt(v + 1e-5) * g_ref[...] + be_ref[...]
        hn = jnp.maximum(hn, 0.0)
        xo = jnp.dot(hn, wb_ref[...], preferred_element_type=jnp.float32)
        xo = xo + bb_ref[...]
        ids = lax.broadcasted_iota(jnp.int32, (B, N), 0)
        oh = (batch_ref[...] == ids).astype(jnp.float32)
        cnt = jnp.maximum(jnp.sum(oh, axis=1, keepdims=True), 1.0)
        pool_ref[...] = jnp.dot(oh, xo, preferred_element_type=jnp.float32) / cnt
        qn_ref[...] = jnp.dot(xo, wna_ref[...],
                              preferred_element_type=jnp.float32)

    return pl.pallas_call(
        body,
        out_shape=[jax.ShapeDtypeStruct((B, H), jnp.float32),
                   jax.ShapeDtypeStruct((N, HP), jnp.float32)],
    )(q, parts, ba, g, be, Wb, bb, Wna, batch2)


def _final(q, parts, ba, g, be, Wb, bb, batch2, pool1, pool2,
           lin1_W, lin1_b, lin2_W, lin2_b):
    def body(q_ref, p_ref, ba_ref, g_ref, be_ref, wb_ref, bb_ref, batch_ref,
             p1_ref, p2_ref, l1w_ref, l1b_ref, l2w_ref, l2b_ref, o_ref):
        h2 = (q_ref[...] + p_ref[0, :N, :] + p_ref[1, :N, :])[:, :H]
        h2 = h2 + ba_ref[...]
        m = jnp.mean(h2, axis=0, keepdims=True)
        v = jnp.mean((h2 - m) * (h2 - m), axis=0, keepdims=True)
        hn = (h2 - m) * lax.rsqrt(v + 1e-5) * g_ref[...] + be_ref[...]
        hn = jnp.maximum(hn, 0.0)
        xo = jnp.dot(hn, wb_ref[...], preferred_element_type=jnp.float32)
        xo = xo + bb_ref[...]
        ids = lax.broadcasted_iota(jnp.int32, (B, N), 0)
        oh = (batch_ref[...] == ids).astype(jnp.float32)
        cnt = jnp.maximum(jnp.sum(oh, axis=1, keepdims=True), 1.0)
        pool3 = jnp.dot(oh, xo, preferred_element_type=jnp.float32) / cnt
        z = jnp.concatenate([p1_ref[...], p2_ref[...], pool3], axis=1)
        zz = jnp.dot(z, l1w_ref[...], preferred_element_type=jnp.float32)
        zz = jnp.maximum(zz + l1b_ref[...], 0.0)
        o_ref[...] = jnp.dot(zz, l2w_ref[...],
                             preferred_element_type=jnp.float32) + l2b_ref[...]

    return pl.pallas_call(
        body,
        out_shape=jax.ShapeDtypeStruct((B, 1), jnp.float32),
    )(q, parts, ba, g, be, Wb, bb, batch2, pool1, pool2,
      lin1_W, lin1_b, lin2_W, lin2_b)


def kernel(x, edge_index, batch,
           W1a, b1a, g1, be1, W1b, b1b,
           W2a, b2a, g2, be2, W2b, b2b,
           W3a, b3a, g3, be3, W3b, b3b,
           lin1_W, lin1_b, lin2_W, lin2_b):
    src = jnp.concatenate(
        [edge_index[0], jnp.zeros((EPAD,), jnp.int32)]).reshape(NW, C, K)
    dst = jnp.concatenate(
        [edge_index[1], jnp.full((EPAD,), N, jnp.int32)]).reshape(NW, C, K)
    zeros = jnp.zeros((NPAD, HP), jnp.float32)
    batch2 = batch.reshape(1, N)

    def agg(q):
        return _agg_kernel(q, src, dst, zeros)

    r = lambda a: a.reshape(1, -1)
    wpad = lambda w: jnp.concatenate(
        [w, jnp.zeros((w.shape[0], HP - H), jnp.float32)], axis=1)

    q1 = pl.pallas_call(
        _mm_body,
        out_shape=jax.ShapeDtypeStruct((N, HP), jnp.float32))(x, wpad(W1a))
    pool1, q2 = _layer_tail(q1, agg(q1), r(b1a), r(g1), r(be1), W1b, r(b1b),
                            wpad(W2a), batch2)
    pool2, q3 = _layer_tail(q2, agg(q2), r(b2a), r(g2), r(be2), W2b, r(b2b),
                            wpad(W3a), batch2)
    return _final(q3, agg(q3), r(b3a), r(g3), r(be3), W3b, r(b3b), batch2,
                  pool1, pool2, lin1_W, r(lin1_b), lin2_W, r(lin2_b))
